# baseline (speedup 1.0000x reference)
import functools

import jax
import jax.numpy as jnp
from jax.experimental import pallas as pl
from jax.experimental.pallas import tpu as pltpu

_B = 16
_CH = 128


def _fused_mlp_segmax(x_ref, posT_ref, bat_ref, w1_ref, w2_ref,
                      out_ref, *, tile: int, n_tiles: int):
    i = pl.program_id(0)
    nchunk = tile // _CH

    @pl.when(i == 0)
    def _init():
        out_ref[:] = jnp.full_like(out_ref, -jnp.inf)

    h = jnp.dot(x_ref[:], w1_ref[:], preferred_element_type=jnp.float32)
    h = h + jax.lax.dot_general(
        posT_ref[:], w2_ref[:],
        dimension_numbers=(((0,), (0,)), ((), ())),
        preferred_element_type=jnp.float32)

    chmax = jnp.max(h.reshape(nchunk, _CH, 128), axis=1)

    bt = bat_ref[0]
    first = bt[0, 0]
    last = bt[7, tile // 8 - 1]

    cstart = jax.lax.broadcasted_iota(jnp.int32, (nchunk, 1), 0) * _CH
    rowf = jax.lax.broadcasted_iota(jnp.int32, (_CH, 1), 0)
    for s in range(_B):
        @pl.when(jnp.logical_and(first <= s, s <= last))
        def _seg(s=s):
            lo = jnp.sum((bt < s).astype(jnp.int32))
            hi = jnp.sum((bt <= s).astype(jnp.int32))
            mc = jnp.logical_and(cstart >= lo, cstart + _CH <= hi)
            seg = jnp.max(jnp.where(mc, chmax, -jnp.inf), axis=0,
                          keepdims=True)
            for edge in (lo, jnp.maximum(hi, 1) - 1):
                a = edge // _CH * _CH
                hb = jnp.dot(x_ref[pl.ds(a, _CH), :], w1_ref[:],
                             preferred_element_type=jnp.float32)
                hb = hb + jax.lax.dot_general(
                    posT_ref[:, pl.ds(a, _CH)], w2_ref[:],
                    dimension_numbers=(((0,), (0,)), ((), ())),
                    preferred_element_type=jnp.float32)
                mb = jnp.logical_and(rowf + a >= lo, rowf + a < hi)
                segb = jnp.max(jnp.where(mb, hb, -jnp.inf), axis=0,
                               keepdims=True)
                seg = jnp.maximum(seg, segb)
            out_ref[s:s + 1, :] = jnp.maximum(out_ref[s:s + 1, :], seg)

    @pl.when(i == n_tiles - 1)
    def _fixup():
        acc = out_ref[:]
        out_ref[:] = jnp.where(acc == -jnp.inf, acc, jnp.maximum(acc, 0.0))


def kernel(x, pos, batch, W, b):
    n, d = x.shape
    tile = 20480
    n_tiles = (n + tile - 1) // tile
    n_pad = n_tiles * tile - n

    w1 = W[:d]
    w2 = jnp.concatenate([W[d:], b.reshape(1, d)])
    posT = jnp.pad(
        jnp.concatenate([pos.T, jnp.ones((1, n), pos.dtype)], axis=0),
        ((0, 0), (0, n_pad)))
    bat3 = jnp.pad(batch.astype(jnp.int32), (0, n_pad),
                   constant_values=_B).reshape(n_tiles, 8, tile // 8)

    pooled = pl.pallas_call(
        functools.partial(_fused_mlp_segmax, tile=tile, n_tiles=n_tiles),
        grid=(n_tiles,),
        in_specs=[
            pl.BlockSpec((tile, d), lambda i: (i, 0)),
            pl.BlockSpec((4, tile), lambda i: (0, i)),
            pl.BlockSpec((1, 8, tile // 8), lambda i: (i, 0, 0)),
            pl.BlockSpec((d, d), lambda i: (0, 0)),
            pl.BlockSpec((4, d), lambda i: (0, 0)),
        ],
        out_specs=pl.BlockSpec((_B, d), lambda i: (0, 0)),
        out_shape=jax.ShapeDtypeStruct((_B, d), jnp.float32),
    )(x, posT, bat3, w1, w2)

    pos_out = jnp.zeros((_B, 3), dtype=pos.dtype)
    batch_out = jnp.arange(_B, dtype=jnp.int64)
    return (pooled, pos_out, batch_out)


# speedup vs baseline: 1.1280x; 1.1280x over previous
import functools

import jax
import jax.numpy as jnp
from jax.experimental import pallas as pl
from jax.experimental.pallas import tpu as pltpu

_B = 16
_CH = 128


def _fused_mlp_segmax(x_ref, posT_ref, bat_ref, w1_ref, w2_ref,
                      out_ref, hs_ref, *, tile: int, n_tiles: int):
    i = pl.program_id(0)
    nchunk = tile // _CH

    @pl.when(i == 0)
    def _init():
        out_ref[:] = jnp.full_like(out_ref, -jnp.inf)

    h = jnp.dot(x_ref[:], w1_ref[:], preferred_element_type=jnp.float32)
    h = h + jax.lax.dot_general(
        posT_ref[:], w2_ref[:],
        dimension_numbers=(((0,), (0,)), ((), ())),
        preferred_element_type=jnp.float32)
    hs_ref[:] = h

    chmax = jnp.max(h.reshape(nchunk, _CH, 128), axis=1)

    bt = bat_ref[0]
    first = bt[0, 0]
    last = bt[7, tile // 8 - 1]

    cstart = jax.lax.broadcasted_iota(jnp.int32, (nchunk, 1), 0) * _CH
    rowf = jax.lax.broadcasted_iota(jnp.int32, (_CH, 1), 0)
    for s in range(_B):
        @pl.when(jnp.logical_and(first <= s, s <= last))
        def _seg(s=s):
            lo = jnp.sum((bt < s).astype(jnp.int32))
            hi = jnp.sum((bt <= s).astype(jnp.int32))
            mc = jnp.logical_and(cstart >= lo, cstart + _CH <= hi)
            seg = jnp.max(jnp.where(mc, chmax, -jnp.inf), axis=0,
                          keepdims=True)
            for edge in (lo, jnp.maximum(hi, 1) - 1):
                a = edge // _CH * _CH
                hb = hs_ref[pl.ds(a, _CH), :]
                mb = jnp.logical_and(rowf + a >= lo, rowf + a < hi)
                segb = jnp.max(jnp.where(mb, hb, -jnp.inf), axis=0,
                               keepdims=True)
                seg = jnp.maximum(seg, segb)
            out_ref[s:s + 1, :] = jnp.maximum(out_ref[s:s + 1, :], seg)

    @pl.when(i == n_tiles - 1)
    def _fixup():
        acc = out_ref[:]
        out_ref[:] = jnp.where(acc == -jnp.inf, acc, jnp.maximum(acc, 0.0))


def kernel(x, pos, batch, W, b):
    n, d = x.shape
    tile = 20096
    n_tiles = (n + tile - 1) // tile
    n_pad = n_tiles * tile - n

    w1 = W[:d]
    w2 = jnp.concatenate([W[d:], b.reshape(1, d)])
    posT = jnp.pad(
        jnp.concatenate([pos.T, jnp.ones((1, n), pos.dtype)], axis=0),
        ((0, 0), (0, n_pad)))
    bat3 = jnp.pad(batch.astype(jnp.int32), (0, n_pad),
                   constant_values=_B).reshape(n_tiles, 8, tile // 8)

    pooled = pl.pallas_call(
        functools.partial(_fused_mlp_segmax, tile=tile, n_tiles=n_tiles),
        grid=(n_tiles,),
        in_specs=[
            pl.BlockSpec((tile, d), lambda i: (i, 0)),
            pl.BlockSpec((4, tile), lambda i: (0, i)),
            pl.BlockSpec((1, 8, tile // 8), lambda i: (i, 0, 0)),
            pl.BlockSpec((d, d), lambda i: (0, 0)),
            pl.BlockSpec((4, d), lambda i: (0, 0)),
        ],
        out_specs=pl.BlockSpec((_B, d), lambda i: (0, 0)),
        out_shape=jax.ShapeDtypeStruct((_B, d), jnp.float32),
        scratch_shapes=[pltpu.VMEM((tile, d), jnp.float32)],
    )(x, posT, bat3, w1, w2)

    pos_out = jnp.zeros((_B, 3), dtype=pos.dtype)
    batch_out = jnp.arange(_B, dtype=jnp.int64)
    return (pooled, pos_out, batch_out)
